# baseline (speedup 1.0000x reference)
import functools

import jax
import jax.numpy as jnp
from jax import lax
from jax.experimental import pallas as pl
from jax.experimental.pallas import tpu as pltpu
from jax.experimental.pallas import tpu_sc as plsc

F = 256
HALF = F // 2
TRIL = F * (F + 1) // 2
S_HALF = HALF * (HALF + 1) // 2
IN_LO = S_HALF
IN_HI = TRIL - S_HALF
OUT = F * F
OUT_HALF = OUT // 2
DIAG_OFFSET = 0.5
NC = 2
NS = 16
NW = NC * NS


def _sc_body(in_hbm, out_hbm, outb, inb, sem_in_hi, sem_in_lo,
             sem_out_hi, sem_out_lo):
    cid = lax.axis_index("c")
    sid = lax.axis_index("s")
    wid = sid * NC + cid
    batch = in_hbm.shape[0] // TRIL
    per_w = batch // NW
    base = wid * per_w

    iota16 = lax.iota(jnp.int32, 16)
    zeros16 = jnp.zeros((16,), jnp.float32)

    def in_hi_copy(b):
        return pltpu.make_async_copy(
            in_hbm.at[pl.ds(b * TRIL + S_HALF, IN_HI)],
            inb.at[pl.ds(S_HALF, IN_HI)], sem_in_hi)

    def in_lo_copy(b):
        return pltpu.make_async_copy(
            in_hbm.at[pl.ds(b * TRIL, IN_LO)], inb.at[pl.ds(0, IN_LO)],
            sem_in_lo)

    def out_hi_copy(b):
        return pltpu.make_async_copy(
            outb.at[pl.ds(OUT_HALF, OUT_HALF)],
            out_hbm.at[pl.ds(b * OUT + OUT_HALF, OUT_HALF)], sem_out_hi)

    def out_lo_copy(b):
        return pltpu.make_async_copy(
            outb.at[pl.ds(0, OUT_HALF)],
            out_hbm.at[pl.ds(b * OUT, OUT_HALF)], sem_out_lo)

    def expand(r0, r1):
        for j in range(F // 16):
            col = j * 16
            lo = max(col + 16, r0)
            if lo >= r1:
                continue

            @plsc.parallel_loop(lo, r1, unroll=8)
            def _copy(r):
                s = (r * (r + 1)) >> 1
                outb[pl.ds(r * F + col, 16)] = inb[pl.ds(s + col, 16)]

        @plsc.parallel_loop(r0, r1, unroll=4)
        def _diag(r):
            s = (r * (r + 1)) >> 1
            jd16 = (r >> 4) * 16
            c = jd16 + iota16
            vals = inb[pl.ds(s + jd16, 16)]
            dval = jnp.abs(DIAG_OFFSET + vals) + 1e-9
            res = jnp.where(c < r, vals, jnp.where(c == r, dval, zeros16))
            outb[pl.ds(r * F + jd16, 16)] = res

    @plsc.parallel_loop(0, OUT // 16, unroll=8)
    def _zero(k):
        outb[pl.ds(k * 16, 16)] = zeros16

    def batch_body(t, _):
        b = base + t
        pltpu.sync_copy(in_hbm.at[pl.ds(b * TRIL, TRIL)], inb)
        expand(HALF, F)
        expand(0, HALF)
        pltpu.sync_copy(outb, out_hbm.at[pl.ds(b * OUT, OUT)])
        return 0
    lax.fori_loop(0, per_w, batch_body, 0)


def kernel(input):
    batch = input.shape[0]
    mesh = plsc.VectorSubcoreMesh(core_axis_name="c", subcore_axis_name="s")
    run = functools.partial(
        pl.kernel,
        mesh=mesh,
        out_type=jax.ShapeDtypeStruct((batch * OUT,), jnp.float32),
        scratch_types=[
            pltpu.VMEM((OUT,), jnp.float32),
            pltpu.VMEM((TRIL,), jnp.float32),
            pltpu.SemaphoreType.DMA,
            pltpu.SemaphoreType.DMA,
            pltpu.SemaphoreType.DMA,
            pltpu.SemaphoreType.DMA,
        ],
    )(_sc_body)
    flat = run(input.reshape(-1))
    return flat.reshape(batch, F, F)


# speedup vs baseline: 1.7700x; 1.7700x over previous
import functools

import jax
import jax.numpy as jnp
from jax import lax
from jax.experimental import pallas as pl
from jax.experimental.pallas import tpu as pltpu
from jax.experimental.pallas import tpu_sc as plsc

F = 256
HALF = F // 2
TRIL = F * (F + 1) // 2
S_HALF = HALF * (HALF + 1) // 2
IN_LO = S_HALF
IN_HI = TRIL - S_HALF
OUT = F * F
OUT_HALF = OUT // 2
DIAG_OFFSET = 0.5
NC = 2
NS = 16
NW = NC * NS


def _sc_body(in_hbm, out_hbm, outb, inb):
    cid = lax.axis_index("c")
    sid = lax.axis_index("s")
    wid = sid * NC + cid
    batch = in_hbm.shape[0] // TRIL
    per_w = batch // NW
    base = wid * per_w

    iota16 = lax.iota(jnp.int32, 16)
    zeros16 = jnp.zeros((16,), jnp.float32)

    def expand(r0, r1):
        for j in range(F // 16):
            col = j * 16
            lo = max(col + 16, r0)
            if lo >= r1:
                continue

            @plsc.parallel_loop(lo, r1, unroll=8)
            def _copy(r):
                s = (r * (r + 1)) >> 1
                outb[r, pl.ds(col, 16)] = inb[pl.ds(s + col, 16)]

        @plsc.parallel_loop(r0, r1, unroll=4)
        def _diag(r):
            s = (r * (r + 1)) >> 1
            jd16 = (r >> 4) * 16
            c = jd16 + iota16
            vals = inb[pl.ds(s + jd16, 16)]
            dval = jnp.abs(DIAG_OFFSET + vals) + 1e-9
            res = jnp.where(c < r, vals, jnp.where(c == r, dval, zeros16))
            outb[r, pl.ds(jd16, 16)] = res

    for j in range(F // 16):
        col = j * 16

        @plsc.parallel_loop(0, F, unroll=8)
        def _zero(r):
            outb[r, pl.ds(col, 16)] = zeros16

    def batch_body(t, _):
        b = base + t
        pltpu.sync_copy(in_hbm.at[pl.ds(b * TRIL, TRIL)], inb)
        expand(HALF, F)
        expand(0, HALF)
        pltpu.sync_copy(outb, out_hbm.at[b])
        return 0
    lax.fori_loop(0, per_w, batch_body, 0)


def kernel(input):
    batch = input.shape[0]
    mesh = plsc.VectorSubcoreMesh(core_axis_name="c", subcore_axis_name="s")
    run = functools.partial(
        pl.kernel,
        mesh=mesh,
        out_type=jax.ShapeDtypeStruct((batch, F, F), jnp.float32),
        scratch_types=[
            pltpu.VMEM((F, F), jnp.float32),
            pltpu.VMEM((TRIL,), jnp.float32),
        ],
    )(_sc_body)
    return run(input.reshape(-1))


# speedup vs baseline: 3.6742x; 2.0758x over previous
import functools

import jax
import jax.numpy as jnp
from jax import lax
from jax.experimental import pallas as pl
from jax.experimental.pallas import tpu as pltpu
from jax.experimental.pallas import tpu_sc as plsc

F = 256
HALF = F // 2
TRIL = F * (F + 1) // 2
DIAG_OFFSET = 0.5
NC = 2
NS = 16
NW = NC * NS


def _sc_body(in_hbm, out_hbm, outb, inb, sem_in, sem_out_hi, sem_out_lo):
    cid = lax.axis_index("c")
    sid = lax.axis_index("s")
    wid = sid * NC + cid
    batch = in_hbm.shape[0]
    per_w = batch // NW
    base = wid * per_w

    iota16 = lax.iota(jnp.int32, 16)
    zeros16 = jnp.zeros((16,), jnp.float32)

    def in_copy(b):
        return pltpu.make_async_copy(in_hbm.at[b], inb, sem_in)

    def out_hi_copy(b):
        return pltpu.make_async_copy(
            outb.at[pl.ds(HALF, HALF)],
            out_hbm.at[b, pl.ds(HALF, HALF)], sem_out_hi)

    def out_lo_copy(b):
        return pltpu.make_async_copy(
            outb.at[pl.ds(0, HALF)],
            out_hbm.at[b, pl.ds(0, HALF)], sem_out_lo)

    def expand(r0, r1):
        for j in range(F // 16):
            col = j * 16
            lo = max(col + 16, r0)
            if lo >= r1:
                continue

            @plsc.parallel_loop(lo, r1, unroll=8)
            def _copy(r):
                s = (r * (r + 1)) >> 1
                outb[r, pl.ds(col, 16)] = inb[pl.ds(s + col, 16)]

        @plsc.parallel_loop(r0, r1, unroll=4)
        def _diag(r):
            s = (r * (r + 1)) >> 1
            jd16 = (r >> 4) * 16
            c = jd16 + iota16
            vals = inb[pl.ds(s + jd16, 16)]
            dval = jnp.abs(DIAG_OFFSET + vals) + 1e-9
            res = jnp.where(c < r, vals, jnp.where(c == r, dval, zeros16))
            outb[r, pl.ds(jd16, 16)] = res

    for j in range(F // 16):
        col = j * 16

        @plsc.parallel_loop(0, F, unroll=8)
        def _zero(r):
            outb[r, pl.ds(col, 16)] = zeros16

    pltpu.sync_copy(in_hbm.at[base], inb)

    def batch_body(t, _):
        b = base + t

        @pl.when(t > 0)
        def _():
            in_copy(b).wait()
            out_hi_copy(b - 1).wait()
        expand(HALF, F)
        out_hi_copy(b).start()

        @pl.when(t > 0)
        def _():
            out_lo_copy(b - 1).wait()
        expand(0, HALF)
        out_lo_copy(b).start()

        @pl.when(t < per_w - 1)
        def _():
            in_copy(b + 1).start()
        return 0
    lax.fori_loop(0, per_w, batch_body, 0)

    out_hi_copy(base + per_w - 1).wait()
    out_lo_copy(base + per_w - 1).wait()


def kernel(input):
    batch = input.shape[0]
    mesh = plsc.VectorSubcoreMesh(core_axis_name="c", subcore_axis_name="s")
    run = functools.partial(
        pl.kernel,
        mesh=mesh,
        out_type=jax.ShapeDtypeStruct((batch, F, F), jnp.float32),
        scratch_types=[
            pltpu.VMEM((F, F), jnp.float32),
            pltpu.VMEM((TRIL,), jnp.float32),
            pltpu.SemaphoreType.DMA,
            pltpu.SemaphoreType.DMA,
            pltpu.SemaphoreType.DMA,
        ],
    )(_sc_body)
    return run(input)


# speedup vs baseline: 3.7945x; 1.0327x over previous
import functools

import jax
import jax.numpy as jnp
from jax import lax
from jax.experimental import pallas as pl
from jax.experimental.pallas import tpu as pltpu
from jax.experimental.pallas import tpu_sc as plsc

F = 256
HALF = F // 2
TRIL = F * (F + 1) // 2
IN_SPLIT = 16384
ROW_SPLIT = 181
DIAG_OFFSET = 0.5
NC = 2
NS = 16
NW = NC * NS


def _sc_body(in_hbm, out_hbm, outb, inb, sem_in1, sem_in2, sem_out_hi,
             sem_out_lo):
    cid = lax.axis_index("c")
    sid = lax.axis_index("s")
    wid = sid * NC + cid
    batch = in_hbm.shape[0]
    per_w = batch // NW
    base = wid * per_w

    iota16 = lax.iota(jnp.int32, 16)
    zeros16 = jnp.zeros((16,), jnp.float32)

    def in1_copy(b):
        return pltpu.make_async_copy(
            in_hbm.at[b, pl.ds(0, IN_SPLIT)],
            inb.at[pl.ds(0, IN_SPLIT)], sem_in1)

    def in2_copy(b):
        return pltpu.make_async_copy(
            in_hbm.at[b, pl.ds(IN_SPLIT, TRIL - IN_SPLIT)],
            inb.at[pl.ds(IN_SPLIT, TRIL - IN_SPLIT)], sem_in2)

    def out_hi_copy(b):
        return pltpu.make_async_copy(
            outb.at[pl.ds(HALF, HALF)],
            out_hbm.at[b, pl.ds(HALF, HALF)], sem_out_hi)

    def out_lo_copy(b):
        return pltpu.make_async_copy(
            outb.at[pl.ds(0, HALF)],
            out_hbm.at[b, pl.ds(0, HALF)], sem_out_lo)

    def expand(r0, r1):
        for j in range(F // 16):
            col = j * 16
            lo = max(col + 16, r0)
            if lo >= r1:
                continue

            @plsc.parallel_loop(lo, r1, unroll=8)
            def _copy(r):
                s = (r * (r + 1)) >> 1
                outb[r, pl.ds(col, 16)] = inb[pl.ds(s + col, 16)]

        @plsc.parallel_loop(r0, r1, unroll=4)
        def _diag(r):
            s = (r * (r + 1)) >> 1
            jd16 = (r >> 4) * 16
            c = jd16 + iota16
            vals = inb[pl.ds(s + jd16, 16)]
            dval = jnp.abs(DIAG_OFFSET + vals) + 1e-9
            res = jnp.where(c < r, vals, jnp.where(c == r, dval, zeros16))
            outb[r, pl.ds(jd16, 16)] = res

    in1_copy(base).start()
    in2_copy(base).start()

    for j in range(F // 16):
        col = j * 16

        @plsc.parallel_loop(0, F, unroll=8)
        def _zero(r):
            outb[r, pl.ds(col, 16)] = zeros16

    in1_copy(base).wait()
    in2_copy(base).wait()

    def batch_body(t, _):
        b = base + t

        @pl.when(t > 0)
        def _():
            in2_copy(b).wait()
            out_hi_copy(b - 1).wait()
        expand(ROW_SPLIT, F)

        @pl.when(t > 0)
        def _():
            in1_copy(b).wait()
        expand(HALF, ROW_SPLIT)
        out_hi_copy(b).start()

        @pl.when(t < per_w - 1)
        def _():
            in2_copy(b + 1).start()

        @pl.when(t > 0)
        def _():
            out_lo_copy(b - 1).wait()
        expand(0, HALF)
        out_lo_copy(b).start()

        @pl.when(t < per_w - 1)
        def _():
            in1_copy(b + 1).start()
        return 0
    lax.fori_loop(0, per_w, batch_body, 0)

    out_hi_copy(base + per_w - 1).wait()
    out_lo_copy(base + per_w - 1).wait()


def kernel(input):
    batch = input.shape[0]
    mesh = plsc.VectorSubcoreMesh(core_axis_name="c", subcore_axis_name="s")
    run = functools.partial(
        pl.kernel,
        mesh=mesh,
        out_type=jax.ShapeDtypeStruct((batch, F, F), jnp.float32),
        scratch_types=[
            pltpu.VMEM((F, F), jnp.float32),
            pltpu.VMEM((TRIL,), jnp.float32),
            pltpu.SemaphoreType.DMA,
            pltpu.SemaphoreType.DMA,
            pltpu.SemaphoreType.DMA,
            pltpu.SemaphoreType.DMA,
        ],
    )(_sc_body)
    return run(input)
